# baseline (speedup 1.0000x reference)
import functools

import numpy as np
import jax
import jax.numpy as jnp
from jax import lax
from jax.experimental import pallas as pl
from jax.experimental.pallas import tpu as pltpu
from jax.experimental.pallas import tpu_sc as plsc

BATCH = 128
NUM_BIN = 8192
BIN_SIZE = np.float32(2.0 / NUM_BIN)
NC, NS, LANES = 2, 16, 16
NW = NC * NS
RPW = BATCH // NW
RACKS = 4
STEPS = NUM_BIN // (LANES * RACKS)


def _threefry2x32(k1, k2, x0, x1):
    u32 = np.uint32
    rot = lambda x, d: (x << u32(d)) | (x >> u32(32 - d))
    ks = [u32(k1), u32(k2), u32(k1) ^ u32(k2) ^ u32(0x1BD11BDA)]
    rots = ([13, 15, 26, 6], [17, 29, 16, 24])
    x0 = x0 + ks[0]
    x1 = x1 + ks[1]
    sched = [(0, ks[1], ks[2]), (1, ks[2], ks[0]), (0, ks[0], ks[1]),
             (1, ks[1], ks[2]), (0, ks[2], ks[0])]
    for i, (grp, a0, a1) in enumerate(sched):
        for r in rots[grp]:
            x0 = x0 + x1
            x1 = rot(x1, r)
            x1 = x0 ^ x1
        x0 = x0 + a0
        x1 = x1 + a1 + u32(i + 1)
    return x0, x1


def _bits_to_unit_float(bits):
    fb = (bits >> np.uint32(9)) | np.uint32(0x3F800000)
    return fb.view(np.float32) - np.float32(1.0)


def _constants():
    u32 = np.uint32
    tiny = np.float32(np.finfo(np.float32).tiny)
    s0, s1 = _threefry2x32(u32(0), u32(42),
                           np.zeros(2, u32), np.arange(2, dtype=u32))
    ky = (s0[0], s1[0])
    ku = (s0[1], s1[1])
    n = BATCH * NUM_BIN
    b0, b1 = _threefry2x32(ky[0], ky[1],
                           np.zeros(n, u32), np.arange(n, dtype=u32))
    uy = _bits_to_unit_float(b0 ^ b1)
    uy = np.maximum(tiny, uy + tiny).reshape(BATCH, NUM_BIN)
    c0, c1 = _threefry2x32(ku[0], ku[1],
                           np.zeros(BATCH, u32), np.arange(BATCH, dtype=u32))
    uu = np.maximum(np.float32(0.0),
                    _bits_to_unit_float(c0 ^ c1) * BIN_SIZE)
    e = (-1.0 / np.log(uy.astype(np.float64))).astype(np.float32)
    u_pad = np.zeros((NW, LANES), np.float32)
    u_pad[:, :RPW] = uu.reshape(NW, RPW)
    return e, u_pad


_E_CONST, _U_PAD = _constants()


@functools.cache
def _build_sampler():
    mesh = plsc.VectorSubcoreMesh(core_axis_name="c", subcore_axis_name="s")
    return pl.kernel(
        _sc_sampler_body,
        out_type=jax.ShapeDtypeStruct((NW, LANES), jnp.float32),
        mesh=mesh,
        scratch_types=[
            pltpu.VMEM((RPW, NUM_BIN), jnp.float32),
            pltpu.VMEM((RPW, NUM_BIN), jnp.float32),
            pltpu.VMEM((LANES,), jnp.float32),
            pltpu.VMEM((LANES,), jnp.float32),
            pltpu.SemaphoreType.DMA,
            pltpu.SemaphoreType.DMA,
            pltpu.SemaphoreType.DMA,
            pltpu.SemaphoreType.DMA,
            pltpu.SemaphoreType.DMA,
        ],
    )


def _sc_sampler_body(p_hbm, e_hbm, u_hbm, out_hbm, pbuf, ebuf, ubuf, sbuf,
                     su, s0, s1, s2, s3):
    wid = lax.axis_index("s") * NC + lax.axis_index("c")
    r0 = wid * RPW
    sems = [s0, s1, s2, s3]

    cu = pltpu.async_copy(u_hbm.at[wid], ubuf, su)
    cps = [pltpu.async_copy(p_hbm.at[r0 + k], pbuf.at[k], sems[k])
           for k in range(RPW)]
    ces = [pltpu.async_copy(e_hbm.at[r0 + k], ebuf.at[k], sems[k])
           for k in range(RPW)]

    lane = jnp.arange(LANES, dtype=jnp.int32)
    yv = jnp.zeros((LANES,), jnp.int32)

    for k in range(RPW):
        cps[k].wait()
        ces[k].wait()

        def body(t, carry, k=k):
            bvs, bjs = carry
            nbv, nbj = [], []
            for r in range(RACKS):
                c = t * RACKS + r
                off = c * LANES
                pv = pbuf[k, pl.ds(off, LANES)]
                ev = ebuf[k, pl.ds(off, LANES)]
                m = (pv + jnp.float32(1e-20)) * ev
                pred = m > bvs[r]
                nbv.append(jnp.where(pred, m, bvs[r]))
                nbj.append(jnp.where(pred, lax.broadcast(c, (LANES,)), bjs[r]))
            return tuple(nbv), tuple(nbj)

        init = (tuple(jnp.full((LANES,), -1.0, jnp.float32) for _ in range(RACKS)),
                tuple(jnp.zeros((LANES,), jnp.int32) for _ in range(RACKS)))
        bvs, bjs = lax.fori_loop(0, STEPS, body, init)

        bv, bj = bvs[0], bjs[0]
        for r in range(1, RACKS):
            v, j = bvs[r], bjs[r]
            take = (v > bv) | ((v == bv) & (j < bj))
            bv = jnp.where(take, v, bv)
            bj = jnp.where(take, j, bj)

        g = bj * LANES + lane
        for s in (1, 2, 4, 8):
            perm = jnp.bitwise_xor(lane, jnp.int32(s))
            ov = bv.at[perm].get(mode="promise_in_bounds")
            og = g.at[perm].get(mode="promise_in_bounds")
            take = (ov > bv) | ((ov == bv) & (og < g))
            bv = jnp.where(take, ov, bv)
            g = jnp.where(take, og, g)
        yv = jnp.where(lane == k, g, yv)

    cu.wait()
    uv = ubuf[...]
    sbuf[...] = (yv.astype(jnp.float32) * BIN_SIZE + uv) - jnp.float32(1.0)
    pltpu.sync_copy(sbuf, out_hbm.at[wid])


def kernel(p):
    out = _build_sampler()(p, jnp.asarray(_E_CONST), jnp.asarray(_U_PAD))
    return jnp.reshape(out[:, :RPW], (-1, 1, 1, 1))


# speedup vs baseline: 1.0030x; 1.0030x over previous
import functools

import numpy as np
import jax
import jax.numpy as jnp
from jax import lax
from jax.experimental import pallas as pl
from jax.experimental.pallas import tpu as pltpu
from jax.experimental.pallas import tpu_sc as plsc

BATCH = 128
NUM_BIN = 8192
BIN_SIZE = np.float32(2.0 / NUM_BIN)
NC, NS, LANES = 2, 16, 16
NW = NC * NS
RPW = BATCH // NW
RACKS = 4
STEPS = NUM_BIN // (LANES * RACKS)


def _threefry2x32(k1, k2, x0, x1):
    u32 = np.uint32
    rot = lambda x, d: (x << u32(d)) | (x >> u32(32 - d))
    ks = [u32(k1), u32(k2), u32(k1) ^ u32(k2) ^ u32(0x1BD11BDA)]
    rots = ([13, 15, 26, 6], [17, 29, 16, 24])
    x0 = x0 + ks[0]
    x1 = x1 + ks[1]
    sched = [(0, ks[1], ks[2]), (1, ks[2], ks[0]), (0, ks[0], ks[1]),
             (1, ks[1], ks[2]), (0, ks[2], ks[0])]
    for i, (grp, a0, a1) in enumerate(sched):
        for r in rots[grp]:
            x0 = x0 + x1
            x1 = rot(x1, r)
            x1 = x0 ^ x1
        x0 = x0 + a0
        x1 = x1 + a1 + u32(i + 1)
    return x0, x1


def _bits_to_unit_float(bits):
    fb = (bits >> np.uint32(9)) | np.uint32(0x3F800000)
    return fb.view(np.float32) - np.float32(1.0)


def _constants():
    u32 = np.uint32
    tiny = np.float32(np.finfo(np.float32).tiny)
    s0, s1 = _threefry2x32(u32(0), u32(42),
                           np.zeros(2, u32), np.arange(2, dtype=u32))
    ky = (s0[0], s1[0])
    ku = (s0[1], s1[1])
    n = BATCH * NUM_BIN
    b0, b1 = _threefry2x32(ky[0], ky[1],
                           np.zeros(n, u32), np.arange(n, dtype=u32))
    uy = _bits_to_unit_float(b0 ^ b1)
    uy = np.maximum(tiny, uy + tiny).reshape(BATCH, NUM_BIN)
    c0, c1 = _threefry2x32(ku[0], ku[1],
                           np.zeros(BATCH, u32), np.arange(BATCH, dtype=u32))
    uu = np.maximum(np.float32(0.0),
                    _bits_to_unit_float(c0 ^ c1) * BIN_SIZE)
    e = (-1.0 / np.log(uy.astype(np.float64))).astype(np.float32)
    u_pad = np.zeros((NW, LANES), np.float32)
    u_pad[:, :RPW] = uu.reshape(NW, RPW)
    return e, u_pad


_E_CONST, _U_PAD = _constants()


@functools.cache
def _build_sampler():
    mesh = plsc.VectorSubcoreMesh(core_axis_name="c", subcore_axis_name="s")
    return pl.kernel(
        _sc_sampler_body,
        out_type=jax.ShapeDtypeStruct((NW, LANES), jnp.float32),
        mesh=mesh,
        scratch_types=[
            pltpu.VMEM((RPW, NUM_BIN), jnp.float32),
            pltpu.VMEM((RPW, NUM_BIN), jnp.float32),
            pltpu.VMEM((LANES,), jnp.float32),
            pltpu.VMEM((LANES,), jnp.float32),
            pltpu.SemaphoreType.DMA,
            pltpu.SemaphoreType.DMA,
            pltpu.SemaphoreType.DMA,
            pltpu.SemaphoreType.DMA,
            pltpu.SemaphoreType.DMA,
        ],
    )


def _sc_sampler_body(p_hbm, e_hbm, u_hbm, out_hbm, pbuf, ebuf, ubuf, sbuf,
                     su, s0, s1, s2, s3):
    wid = lax.axis_index("s") * NC + lax.axis_index("c")
    r0 = wid * RPW
    sems = [s0, s1, s2, s3]

    cu = pltpu.async_copy(u_hbm.at[wid], ubuf, su)
    cps = [pltpu.async_copy(p_hbm.at[r0 + k], pbuf.at[k], sems[k])
           for k in range(RPW)]
    ces = [pltpu.async_copy(e_hbm.at[pl.ds((r0 + k) * NUM_BIN, NUM_BIN)],
                            ebuf.at[k], sems[k])
           for k in range(RPW)]

    lane = jnp.arange(LANES, dtype=jnp.int32)
    yv = jnp.zeros((LANES,), jnp.int32)

    for k in range(RPW):
        cps[k].wait()
        ces[k].wait()

        def body(t, carry, k=k):
            bvs, bjs = carry
            nbv, nbj = [], []
            for r in range(RACKS):
                c = t * RACKS + r
                off = c * LANES
                pv = pbuf[k, pl.ds(off, LANES)]
                ev = ebuf[k, pl.ds(off, LANES)]
                m = (pv + jnp.float32(1e-20)) * ev
                pred = m > bvs[r]
                nbv.append(jnp.where(pred, m, bvs[r]))
                nbj.append(jnp.where(pred, lax.broadcast(c, (LANES,)), bjs[r]))
            return tuple(nbv), tuple(nbj)

        init = (tuple(jnp.full((LANES,), -1.0, jnp.float32) for _ in range(RACKS)),
                tuple(jnp.zeros((LANES,), jnp.int32) for _ in range(RACKS)))
        bvs, bjs = lax.fori_loop(0, STEPS, body, init)

        bv, bj = bvs[0], bjs[0]
        for r in range(1, RACKS):
            v, j = bvs[r], bjs[r]
            take = (v > bv) | ((v == bv) & (j < bj))
            bv = jnp.where(take, v, bv)
            bj = jnp.where(take, j, bj)

        g = bj * LANES + lane
        for s in (1, 2, 4, 8):
            perm = jnp.bitwise_xor(lane, jnp.int32(s))
            ov = bv.at[perm].get(mode="promise_in_bounds")
            og = g.at[perm].get(mode="promise_in_bounds")
            take = (ov > bv) | ((ov == bv) & (og < g))
            bv = jnp.where(take, ov, bv)
            g = jnp.where(take, og, g)
        yv = jnp.where(lane == k, g, yv)

    cu.wait()
    uv = ubuf[...]
    sbuf[...] = (yv.astype(jnp.float32) * BIN_SIZE + uv) - jnp.float32(1.0)
    pltpu.sync_copy(sbuf, out_hbm.at[wid])


def kernel(p):
    out = _build_sampler()(p, jnp.asarray(_E_CONST.reshape(-1)),
                           jnp.asarray(_U_PAD))
    return jnp.reshape(out[:, :RPW], (-1, 1, 1, 1))


# speedup vs baseline: 1.0077x; 1.0047x over previous
import functools

import numpy as np
import jax
import jax.numpy as jnp
from jax import lax
from jax.experimental import pallas as pl
from jax.experimental.pallas import tpu as pltpu
from jax.experimental.pallas import tpu_sc as plsc

BATCH = 128
NUM_BIN = 8192
BIN_SIZE = np.float32(2.0 / NUM_BIN)
NC, NS, LANES = 2, 16, 16
NW = NC * NS
RPW = BATCH // NW
RACKS = 4
STEPS = NUM_BIN // (LANES * RACKS)


def _threefry2x32(k1, k2, x0, x1):
    u32 = np.uint32
    rot = lambda x, d: (x << u32(d)) | (x >> u32(32 - d))
    ks = [u32(k1), u32(k2), u32(k1) ^ u32(k2) ^ u32(0x1BD11BDA)]
    rots = ([13, 15, 26, 6], [17, 29, 16, 24])
    x0 = x0 + ks[0]
    x1 = x1 + ks[1]
    sched = [(0, ks[1], ks[2]), (1, ks[2], ks[0]), (0, ks[0], ks[1]),
             (1, ks[1], ks[2]), (0, ks[2], ks[0])]
    for i, (grp, a0, a1) in enumerate(sched):
        for r in rots[grp]:
            x0 = x0 + x1
            x1 = rot(x1, r)
            x1 = x0 ^ x1
        x0 = x0 + a0
        x1 = x1 + a1 + u32(i + 1)
    return x0, x1


def _bits_to_unit_float(bits):
    fb = (bits >> np.uint32(9)) | np.uint32(0x3F800000)
    return fb.view(np.float32) - np.float32(1.0)


def _constants():
    u32 = np.uint32
    tiny = np.float32(np.finfo(np.float32).tiny)
    s0, s1 = _threefry2x32(u32(0), u32(42),
                           np.zeros(2, u32), np.arange(2, dtype=u32))
    ky = (s0[0], s1[0])
    ku = (s0[1], s1[1])
    n = BATCH * NUM_BIN
    b0, b1 = _threefry2x32(ky[0], ky[1],
                           np.zeros(n, u32), np.arange(n, dtype=u32))
    uy = _bits_to_unit_float(b0 ^ b1)
    uy = np.maximum(tiny, uy + tiny).reshape(BATCH, NUM_BIN)
    c0, c1 = _threefry2x32(ku[0], ku[1],
                           np.zeros(BATCH, u32), np.arange(BATCH, dtype=u32))
    uu = np.maximum(np.float32(0.0),
                    _bits_to_unit_float(c0 ^ c1) * BIN_SIZE)
    e = (-1.0 / np.log(uy.astype(np.float64))).astype(np.float32)
    u_pad = np.zeros((NW, LANES), np.float32)
    u_pad[:, :RPW] = uu.reshape(NW, RPW)
    return e, u_pad


_E_CONST, _U_PAD = _constants()


@functools.cache
def _build_sampler():
    mesh = plsc.VectorSubcoreMesh(core_axis_name="c", subcore_axis_name="s")
    return pl.kernel(
        _sc_sampler_body,
        out_type=jax.ShapeDtypeStruct((NW, LANES), jnp.float32),
        mesh=mesh,
        scratch_types=[
            pltpu.VMEM((RPW, NUM_BIN), jnp.float32),
            pltpu.VMEM((RPW, NUM_BIN), jnp.float32),
            pltpu.VMEM((LANES,), jnp.float32),
            pltpu.VMEM((LANES,), jnp.float32),
            pltpu.SemaphoreType.DMA,
            pltpu.SemaphoreType.DMA,
            pltpu.SemaphoreType.DMA,
            pltpu.SemaphoreType.DMA,
            pltpu.SemaphoreType.DMA,
        ],
        compiler_params=pltpu.CompilerParams(use_tc_tiling_on_sc=True),
    )


def _sc_sampler_body(p_hbm, e_hbm, u_hbm, out_hbm, pbuf, ebuf, ubuf, sbuf,
                     su, s0, s1, s2, s3):
    wid = lax.axis_index("s") * NC + lax.axis_index("c")
    r0 = wid * RPW
    sems = [s0, s1, s2, s3]

    cu = pltpu.async_copy(u_hbm.at[wid], ubuf, su)
    cps = [pltpu.async_copy(p_hbm.at[r0 + k], pbuf.at[k], sems[k])
           for k in range(RPW)]
    ces = [pltpu.async_copy(e_hbm.at[pl.ds((r0 + k) * NUM_BIN, NUM_BIN)],
                            ebuf.at[k], sems[k])
           for k in range(RPW)]

    lane = jnp.arange(LANES, dtype=jnp.int32)
    yv = jnp.zeros((LANES,), jnp.int32)

    for k in range(RPW):
        cps[k].wait()
        ces[k].wait()

        def body(t, carry, k=k):
            bvs, bjs = carry
            nbv, nbj = [], []
            for r in range(RACKS):
                c = t * RACKS + r
                off = c * LANES
                pv = pbuf[k, pl.ds(off, LANES)]
                ev = ebuf[k, pl.ds(off, LANES)]
                m = (pv + jnp.float32(1e-20)) * ev
                pred = m > bvs[r]
                nbv.append(jnp.where(pred, m, bvs[r]))
                nbj.append(jnp.where(pred, lax.broadcast(c, (LANES,)), bjs[r]))
            return tuple(nbv), tuple(nbj)

        init = (tuple(jnp.full((LANES,), -1.0, jnp.float32) for _ in range(RACKS)),
                tuple(jnp.zeros((LANES,), jnp.int32) for _ in range(RACKS)))
        bvs, bjs = lax.fori_loop(0, STEPS, body, init)

        bv, bj = bvs[0], bjs[0]
        for r in range(1, RACKS):
            v, j = bvs[r], bjs[r]
            take = (v > bv) | ((v == bv) & (j < bj))
            bv = jnp.where(take, v, bv)
            bj = jnp.where(take, j, bj)

        g = bj * LANES + lane
        for s in (1, 2, 4, 8):
            perm = jnp.bitwise_xor(lane, jnp.int32(s))
            ov = bv.at[perm].get(mode="promise_in_bounds")
            og = g.at[perm].get(mode="promise_in_bounds")
            take = (ov > bv) | ((ov == bv) & (og < g))
            bv = jnp.where(take, ov, bv)
            g = jnp.where(take, og, g)
        yv = jnp.where(lane == k, g, yv)

    cu.wait()
    uv = ubuf[...]
    sbuf[...] = (yv.astype(jnp.float32) * BIN_SIZE + uv) - jnp.float32(1.0)
    pltpu.sync_copy(sbuf, out_hbm.at[wid])


def kernel(p):
    out = _build_sampler()(p, jnp.asarray(_E_CONST.reshape(-1)),
                           jnp.asarray(_U_PAD))
    return jnp.reshape(out[:, :RPW], (-1, 1, 1, 1))
